# baseline (speedup 1.0000x reference)
import functools

import jax
import jax.numpy as jnp
from jax import lax
from jax.experimental import pallas as pl
from jax.experimental.pallas import tpu as pltpu
from jax.experimental.pallas import tpu_sc as plsc

_K = 4
_NBUF = 2
_BLK = 256
_NSLOT = 4


def _sc_gather_kernel(nc, bpw, nchunk, table_ref, idx_ref, out_ref, idx_v,
                      rows_v, gsems, wsems):
    w = lax.axis_index("s") * nc + lax.axis_index("c")
    base = w * bpw
    pltpu.sync_copy(idx_ref.at[w], idx_v)

    for b in range(_NBUF):
        pltpu.make_async_copy(
            table_ref.at[idx_v.at[b]], rows_v.at[b], gsems.at[b]
        ).start()

    @pl.loop(0, nchunk, step=_NBUF)
    def _chunks(c):
        for b in range(_NBUF):
            cc = c + b
            pltpu.make_async_copy(
                table_ref.at[idx_v.at[cc]], rows_v.at[b], gsems.at[b]
            ).wait()
            pltpu.make_async_copy(
                rows_v.at[b], out_ref.at[pl.ds(base + cc * _K, _K)],
                wsems.at[b],
            ).start()

            @pl.when(cc + _NBUF < nchunk)
            def _():
                pltpu.make_async_copy(
                    rows_v.at[b], out_ref.at[pl.ds(base + cc * _K, _K)],
                    wsems.at[b],
                ).wait()
                pltpu.make_async_copy(
                    table_ref.at[idx_v.at[cc + _NBUF]], rows_v.at[b],
                    gsems.at[b]
                ).start()

    for b in range(_NBUF):
        cc = nchunk - _NBUF + b
        pltpu.make_async_copy(
            rows_v.at[b], out_ref.at[pl.ds(base + cc * _K, _K)], wsems.at[b]
        ).wait()


def _tc_loss_kernel(nblk, logits_ref, tgt_ref, acc_ref, bufs, tgt_v, sems,
                    tsem):
    pltpu.make_async_copy(tgt_ref, tgt_v, tsem).start()
    for s in range(_NSLOT):
        pltpu.make_async_copy(
            logits_ref.at[pl.ds(s * _BLK, _BLK)], bufs.at[s], sems.at[s]
        ).start()
    pltpu.make_async_copy(tgt_ref, tgt_v, tsem).wait()

    def body(i, acc):
        s = lax.rem(i, _NSLOT)
        pltpu.make_async_copy(
            logits_ref.at[pl.ds(i * _BLK, _BLK)], bufs.at[s], sems.at[s]
        ).wait()
        block = bufs[s]
        m = jnp.max(block, axis=1, keepdims=True)
        e = jnp.sum(jnp.exp(block - m), axis=1, keepdims=True)
        logz = m + jnp.log(e)
        lanes = lax.broadcasted_iota(jnp.int32, block.shape, 1)
        onehot = lanes == tgt_v[pl.ds(i * _BLK, _BLK)]
        picked = jnp.sum(jnp.where(onehot, block, 0.0), axis=1, keepdims=True)
        part = jnp.sum(logz - picked)

        @pl.when(i + _NSLOT < nblk)
        def _():
            pltpu.make_async_copy(
                logits_ref.at[pl.ds((i + _NSLOT) * _BLK, _BLK)], bufs.at[s],
                sems.at[s],
            ).start()

        return acc + part

    acc_ref[...] = lax.fori_loop(
        0, nblk, body, jnp.zeros((1, 1), jnp.float32)
    )


def kernel(idx, targets, table):
    B, T = idx.shape
    V, C = table.shape
    n_tok = B * T
    idx_flat = idx.reshape(n_tok).astype(jnp.int32)
    tgt_flat = targets.reshape(n_tok).astype(jnp.int32)

    info = plsc.get_sparse_core_info()
    nc, ns = info.num_cores, info.num_subcores
    nw = nc * ns
    bpw = n_tok // nw
    nchunk = bpw // _K

    idx3d = idx_flat.reshape(nw, nchunk, _K)

    sc_call = pl.kernel(
        functools.partial(_sc_gather_kernel, nc, bpw, nchunk),
        out_type=jax.ShapeDtypeStruct((n_tok, C), jnp.float32),
        mesh=plsc.VectorSubcoreMesh(
            core_axis_name="c", subcore_axis_name="s"
        ),
        scratch_types=[
            pltpu.VMEM((nchunk, _K), jnp.int32),
            pltpu.VMEM((_NBUF, _K, C), jnp.float32),
            pltpu.SemaphoreType.DMA((_NBUF,)),
            pltpu.SemaphoreType.DMA((_NBUF,)),
        ],
    )
    logits_flat = sc_call(table, idx3d)

    nblk = n_tok // _BLK
    loss_sum = pl.pallas_call(
        functools.partial(_tc_loss_kernel, nblk),
        in_specs=[
            pl.BlockSpec(memory_space=pltpu.HBM),
            pl.BlockSpec(memory_space=pltpu.HBM),
        ],
        out_specs=pl.BlockSpec(memory_space=pltpu.VMEM),
        out_shape=jax.ShapeDtypeStruct((1, 1), jnp.float32),
        scratch_shapes=[
            pltpu.VMEM((_NSLOT, _BLK, C), jnp.float32),
            pltpu.VMEM((n_tok, 1), jnp.int32),
            pltpu.SemaphoreType.DMA((_NSLOT,)),
            pltpu.SemaphoreType.DMA,
        ],
    )(logits_flat, tgt_flat.reshape(n_tok, 1))

    loss = loss_sum[0, 0] / n_tok
    return logits_flat.reshape(B, T, C), loss
